# baseline (speedup 1.0000x reference)
import functools

import jax
import jax.numpy as jnp
from jax import lax
from jax.experimental import pallas as pl
from jax.experimental.pallas import tpu as pltpu
from jax.experimental.pallas import tpu_sc as plsc

_B, _S, _D = 128, 200, 768
_BS = _B * _S
_TB = 512
_GRID = _BS // _TB
_NW = 32
_HALF = _BS // 2
_GRID_H = _GRID // 2
_EPS = 1e-12

_I_MAG, _I_PRE, _I_TOP, _I_LOW, _I_ORD, _I_ROW, _I_COL = range(7)
_I_LT = 7
_I_TT = 11


def _pick_ch(bpw):
    for ch in range(80, 7, -8):
        if bpw % ch == 0:
            return ch
    raise ValueError(bpw)


def _sc_gather(table, idx, off, n):
    bpw = n // _NW
    ch = _pick_ch(bpw)
    nch = bpw // ch
    mesh = plsc.VectorSubcoreMesh(core_axis_name="c", subcore_axis_name="s")

    @functools.partial(
        pl.kernel,
        mesh=mesh,
        out_type=jax.ShapeDtypeStruct((n, _D), jnp.float32),
        scratch_types=[
            pltpu.VMEM((bpw,), jnp.int32),
            pltpu.VMEM((ch, _D), jnp.float32),
            pltpu.VMEM((ch, _D), jnp.float32),
            pltpu.SemaphoreType.DMA,
            pltpu.SemaphoreType.DMA,
            pltpu.SemaphoreType.DMA,
            pltpu.SemaphoreType.DMA,
        ],
    )
    def k(table_hbm, idx_hbm, out_hbm, idx_v, rows_a, rows_b,
          gsem_a, gsem_b, ssem_a, ssem_b):
        wid = lax.axis_index("s") * 2 + lax.axis_index("c")
        base = wid * bpw
        pltpu.sync_copy(idx_hbm.at[pl.ds(off + base, bpw)], idx_v)
        bufs = (rows_a, rows_b)
        gsems = (gsem_a, gsem_b)
        ssems = (ssem_a, ssem_b)
        cps = [None, None]
        sto = [None, None]
        cps[0] = pltpu.async_copy(
            table_hbm.at[idx_v.at[pl.ds(0, ch)]], bufs[0], gsems[0])
        for c in range(nch):
            if c + 1 < nch:
                p = (c + 1) % 2
                if sto[p] is not None:
                    sto[p].wait()
                    sto[p] = None
                cps[p] = pltpu.async_copy(
                    table_hbm.at[idx_v.at[pl.ds((c + 1) * ch, ch)]],
                    bufs[p], gsems[p])
            cps[c % 2].wait()
            sto[c % 2] = pltpu.async_copy(
                bufs[c % 2], out_hbm.at[pl.ds(base + c * ch, ch)],
                ssems[c % 2])
        for s in sto:
            if s is not None:
                s.wait()

    return k(table, idx)


def _tc_compute(tok, magI, preI, topI, lowI, ordI, rowI, colI,
                lt0, lt1, lt2, lt3, tt0, tt1, tt2, tt3, fv,
                denT, rowT, colT, ltT, ttT, g, b, out):
    f32 = jnp.float32

    def dot(a, t):
        return lax.dot_general(a, t, (((1,), (0,)), ((), ())),
                               preferred_element_type=f32)

    def col(iref):
        return iref[0, 0, 0, :].astype(jnp.int32)[:, None]

    def oh(iref, n):
        io = lax.broadcasted_iota(jnp.int32, (_TB, n), 1)
        return (io == col(iref)).astype(jnp.bfloat16)

    io128 = lax.broadcasted_iota(jnp.int32, (_TB, 128), 1)
    sel = jnp.where(io128 < 16, col(magI),
          jnp.where(io128 < 32, col(preI) + 16,
          jnp.where(io128 < 48, col(topI) + 32,
          jnp.where(io128 < 64, col(lowI) + 48, col(ordI) + 64))))
    den = jnp.concatenate(
        [(io128 == sel).astype(jnp.bfloat16), fv[...].astype(jnp.bfloat16)],
        axis=1)
    dense = dot(den, denT[...])

    rows = dot(oh(rowI, 264), rowT[...])
    cols = dot(oh(colI, 264), colT[...])
    lts = [dot(oh(r, 392), ltT[...]) for r in (lt0, lt1, lt2, lt3)]
    tts = [dot(oh(r, 392), ttT[...]) for r in (tt0, tt1, tt2, tt3)]

    pos = jnp.concatenate([rows] + lts + [cols] + tts, axis=1)
    emb = tok[...] + dense + pos
    mu = jnp.mean(emb, axis=1, keepdims=True)
    cen = emb - mu
    var = jnp.mean(cen * cen, axis=1, keepdims=True)
    out[...] = cen * lax.rsqrt(var + _EPS) * g[...] + b[...]


def _tc_body_first(*refs):
    _tc_compute(*refs)


def _tc_body_alias(buf, *refs):
    del buf
    _tc_compute(*refs)


def _tc_fused(tok, idxstack, fv, tables, g, b, off_blocks, nblocks, buf):
    def ispec(k):
        return pl.BlockSpec((1, 1, 1, _TB),
                            lambda i, _k=k, _o=off_blocks: (_k, i + _o, 0, 0))

    def full(shape):
        r = len(shape)
        return pl.BlockSpec(shape, lambda i, _r=r: (0,) * _r)

    in_specs = ([pl.BlockSpec((_TB, _D), lambda i: (i, 0))]
                + [ispec(k) for k in range(15)]
                + [pl.BlockSpec((_TB, 11),
                                lambda i, _o=off_blocks: (i + _o, 0)),
                   full((139, _D)),
                   full((264, 96)), full((264, 96)),
                   full((392, 72)), full((392, 72)),
                   full((1, _D)), full((1, _D))])
    args = (tok,) + (idxstack,) * 15 + (fv,) + tuple(tables) + (g, b)
    if buf is None:
        body = _tc_body_first
        aliases = {}
    else:
        body = _tc_body_alias
        in_specs = [pl.BlockSpec(memory_space=pl.ANY)] + in_specs
        args = (buf,) + args
        aliases = {0: 0}
    return pl.pallas_call(
        body,
        grid=(nblocks,),
        in_specs=in_specs,
        out_specs=pl.BlockSpec((_TB, _D),
                               lambda i, _o=off_blocks: (i + _o, 0)),
        out_shape=jax.ShapeDtypeStruct((_BS, _D), jnp.float32),
        input_output_aliases=aliases,
    )(*args)


def kernel(token_id, num_mag, num_pre, num_top, num_low, order, pos_row,
           pos_col, pos_top, pos_left, format_vec, token_table,
           magnitude_table, precision_table, top_digit_table,
           low_digit_table, order_table, row_table, column_table,
           top_tree_table, left_tree_table, format_W, ln_gamma, ln_beta):
    bf16 = jnp.bfloat16

    tid = token_id.reshape(_BS).astype(jnp.int32)
    pieces = (10, 16, 24)
    toks = []
    off = 0
    for nb in pieces:
        toks.append(_sc_gather(token_table, tid, off * _TB, nb * _TB))
        off += nb

    pt = pos_top.reshape(_BS, 4)
    pf = pos_left.reshape(_BS, 4)
    idxstack = jnp.stack(
        [num_mag.reshape(_BS), num_pre.reshape(_BS), num_top.reshape(_BS),
         num_low.reshape(_BS), order.reshape(_BS), pos_row.reshape(_BS),
         pos_col.reshape(_BS),
         pf[:, 0], pf[:, 1], pf[:, 2], pf[:, 3],
         pt[:, 0], pt[:, 1], pt[:, 2], pt[:, 3]],
        axis=0).astype(jnp.int16).reshape(15, _GRID, 1, _TB)

    numT = jnp.zeros((64, _D), jnp.float32)
    numT = (numT.at[0:12, 0:192].set(magnitude_table)
                .at[16:28, 192:384].set(precision_table)
                .at[32:44, 384:576].set(top_digit_table)
                .at[48:60, 576:768].set(low_digit_table))
    denT = jnp.concatenate(
        [numT, order_table, format_W.T], axis=0).astype(bf16)
    rowT = jnp.pad(row_table, ((0, 7), (0, 0))).astype(bf16)
    colT = jnp.pad(column_table, ((0, 7), (0, 0))).astype(bf16)
    ltT = jnp.pad(left_tree_table, ((0, 7), (0, 0))).astype(bf16)
    ttT = jnp.pad(top_tree_table, ((0, 7), (0, 0))).astype(bf16)
    tables = [denT, rowT, colT, ltT, ttT]
    fv = format_vec.reshape(_BS, 11)
    g2, b2 = ln_gamma.reshape(1, _D), ln_beta.reshape(1, _D)

    buf = None
    off = 0
    for nb, tok in zip(pieces, toks):
        buf = _tc_fused(tok, idxstack, fv, tables, g2, b2, off, nb, buf)
        off += nb
    return buf.reshape(_B, _S, _D)


# speedup vs baseline: 1.0086x; 1.0086x over previous
import functools

import jax
import jax.numpy as jnp
from jax import lax
from jax.experimental import pallas as pl
from jax.experimental.pallas import tpu as pltpu
from jax.experimental.pallas import tpu_sc as plsc

_B, _S, _D = 128, 200, 768
_BS = _B * _S
_TB = 512
_GRID = _BS // _TB
_NW = 32
_HALF = _BS // 2
_GRID_H = _GRID // 2
_EPS = 1e-12

_I_MAG, _I_PRE, _I_TOP, _I_LOW, _I_ORD, _I_ROW, _I_COL = range(7)
_I_LT = 7
_I_TT = 11


def _pick_ch(bpw):
    for ch in range(80, 7, -8):
        if bpw % ch == 0:
            return ch
    raise ValueError(bpw)


def _sc_gather(table, idx, off, n):
    bpw = n // _NW
    ch = _pick_ch(bpw)
    nch = bpw // ch
    mesh = plsc.VectorSubcoreMesh(core_axis_name="c", subcore_axis_name="s")

    @functools.partial(
        pl.kernel,
        mesh=mesh,
        out_type=jax.ShapeDtypeStruct((n, _D), jnp.float32),
        scratch_types=[
            pltpu.VMEM((bpw,), jnp.int32),
            pltpu.VMEM((ch, _D), jnp.float32),
            pltpu.VMEM((ch, _D), jnp.float32),
            pltpu.SemaphoreType.DMA,
            pltpu.SemaphoreType.DMA,
            pltpu.SemaphoreType.DMA,
            pltpu.SemaphoreType.DMA,
        ],
    )
    def k(table_hbm, idx_hbm, out_hbm, idx_v, rows_a, rows_b,
          gsem_a, gsem_b, ssem_a, ssem_b):
        wid = lax.axis_index("s") * 2 + lax.axis_index("c")
        base = wid * bpw
        pltpu.sync_copy(idx_hbm.at[pl.ds(off + base, bpw)], idx_v)
        bufs = (rows_a, rows_b)
        gsems = (gsem_a, gsem_b)
        ssems = (ssem_a, ssem_b)
        cps = [None, None]
        sto = [None, None]
        cps[0] = pltpu.async_copy(
            table_hbm.at[idx_v.at[pl.ds(0, ch)]], bufs[0], gsems[0])
        for c in range(nch):
            if c + 1 < nch:
                p = (c + 1) % 2
                if sto[p] is not None:
                    sto[p].wait()
                    sto[p] = None
                cps[p] = pltpu.async_copy(
                    table_hbm.at[idx_v.at[pl.ds((c + 1) * ch, ch)]],
                    bufs[p], gsems[p])
            cps[c % 2].wait()
            sto[c % 2] = pltpu.async_copy(
                bufs[c % 2], out_hbm.at[pl.ds(base + c * ch, ch)],
                ssems[c % 2])
        for s in sto:
            if s is not None:
                s.wait()

    return k(table, idx)


def _tc_compute(tok, idxs, fv, denT, rowT, colT, ltT, ttT, g, b, out):
    f32 = jnp.float32

    def dot(a, t):
        return lax.dot_general(a, t, (((1,), (0,)), ((), ())),
                               preferred_element_type=f32)

    def col(k):
        return idxs[k, 0, 0, :].astype(jnp.int32)[:, None]

    def oh(k, n):
        io = lax.broadcasted_iota(jnp.int32, (_TB, n), 1)
        return (io == col(k)).astype(jnp.bfloat16)

    io128 = lax.broadcasted_iota(jnp.int32, (_TB, 128), 1)
    sel = jnp.where(io128 < 16, col(_I_MAG),
          jnp.where(io128 < 32, col(_I_PRE) + 16,
          jnp.where(io128 < 48, col(_I_TOP) + 32,
          jnp.where(io128 < 64, col(_I_LOW) + 48, col(_I_ORD) + 64))))
    den = jnp.concatenate(
        [(io128 == sel).astype(jnp.bfloat16), fv[...].astype(jnp.bfloat16)],
        axis=1)
    dense = dot(den, denT[...])

    rows = dot(oh(_I_ROW, 264), rowT[...])
    cols = dot(oh(_I_COL, 264), colT[...])
    lts = [dot(oh(_I_LT + k, 392), ltT[...]) for k in range(4)]
    tts = [dot(oh(_I_TT + k, 392), ttT[...]) for k in range(4)]

    pos = jnp.concatenate([rows] + lts + [cols] + tts, axis=1)
    emb = tok[...] + dense + pos
    mu = jnp.mean(emb, axis=1, keepdims=True)
    cen = emb - mu
    var = jnp.mean(cen * cen, axis=1, keepdims=True)
    out[...] = cen * lax.rsqrt(var + _EPS) * g[...] + b[...]


def _tc_body_first(*refs):
    _tc_compute(*refs)


def _tc_body_alias(buf, *refs):
    del buf
    _tc_compute(*refs)


def _tc_fused(tok, idxstack, fv, tables, g, b, off_blocks, nblocks, buf):
    def full(shape):
        r = len(shape)
        return pl.BlockSpec(shape, lambda i, _r=r: (0,) * _r)

    in_specs = ([pl.BlockSpec((_TB, _D), lambda i: (i, 0)),
                 pl.BlockSpec((15, 1, 1, _TB),
                              lambda i, _o=off_blocks: (0, i + _o, 0, 0)),
                 pl.BlockSpec((_TB, 11),
                              lambda i, _o=off_blocks: (i + _o, 0)),
                 full((139, _D)),
                 full((264, 96)), full((264, 96)),
                 full((392, 72)), full((392, 72)),
                 full((1, _D)), full((1, _D))])
    args = (tok, idxstack, fv) + tuple(tables) + (g, b)
    if buf is None:
        body = _tc_body_first
        aliases = {}
    else:
        body = _tc_body_alias
        in_specs = [pl.BlockSpec(memory_space=pl.ANY)] + in_specs
        args = (buf,) + args
        aliases = {0: 0}
    return pl.pallas_call(
        body,
        grid=(nblocks,),
        in_specs=in_specs,
        out_specs=pl.BlockSpec((_TB, _D),
                               lambda i, _o=off_blocks: (i + _o, 0)),
        out_shape=jax.ShapeDtypeStruct((_BS, _D), jnp.float32),
        input_output_aliases=aliases,
    )(*args)


def kernel(token_id, num_mag, num_pre, num_top, num_low, order, pos_row,
           pos_col, pos_top, pos_left, format_vec, token_table,
           magnitude_table, precision_table, top_digit_table,
           low_digit_table, order_table, row_table, column_table,
           top_tree_table, left_tree_table, format_W, ln_gamma, ln_beta):
    bf16 = jnp.bfloat16

    tid = token_id.reshape(_BS).astype(jnp.int32)
    pieces = (10, 16, 24)
    toks = []
    off = 0
    for nb in pieces:
        toks.append(_sc_gather(token_table, tid, off * _TB, nb * _TB))
        off += nb

    pt = pos_top.reshape(_BS, 4)
    pf = pos_left.reshape(_BS, 4)
    idxstack = jnp.stack(
        [num_mag.reshape(_BS), num_pre.reshape(_BS), num_top.reshape(_BS),
         num_low.reshape(_BS), order.reshape(_BS), pos_row.reshape(_BS),
         pos_col.reshape(_BS),
         pf[:, 0], pf[:, 1], pf[:, 2], pf[:, 3],
         pt[:, 0], pt[:, 1], pt[:, 2], pt[:, 3]],
        axis=0).astype(jnp.int16).reshape(15, _GRID, 1, _TB)

    numT = jnp.zeros((64, _D), jnp.float32)
    numT = (numT.at[0:12, 0:192].set(magnitude_table)
                .at[16:28, 192:384].set(precision_table)
                .at[32:44, 384:576].set(top_digit_table)
                .at[48:60, 576:768].set(low_digit_table))
    denT = jnp.concatenate(
        [numT, order_table, format_W.T], axis=0).astype(bf16)
    rowT = jnp.pad(row_table, ((0, 7), (0, 0))).astype(bf16)
    colT = jnp.pad(column_table, ((0, 7), (0, 0))).astype(bf16)
    ltT = jnp.pad(left_tree_table, ((0, 7), (0, 0))).astype(bf16)
    ttT = jnp.pad(top_tree_table, ((0, 7), (0, 0))).astype(bf16)
    tables = [denT, rowT, colT, ltT, ttT]
    fv = format_vec.reshape(_BS, 11)
    g2, b2 = ln_gamma.reshape(1, _D), ln_beta.reshape(1, _D)

    buf = None
    off = 0
    for nb, tok in zip(pieces, toks):
        buf = _tc_fused(tok, idxstack, fv, tables, g2, b2, off, nb, buf)
        off += nb
    return buf.reshape(_B, _S, _D)
